# Initial kernel scaffold; baseline (speedup 1.0000x reference)
import jax
import jax.numpy as jnp
from jax.experimental import pallas as pl


def kernel(x, node_index, edge_index, edge_attr, params):
    raise NotImplementedError("write your pallas kernel here")


# speedup vs baseline: 4.4014x; 4.4014x over previous
import functools

import jax
import jax.numpy as jnp
from jax import lax
from jax.experimental import pallas as pl
from jax.experimental.pallas import tpu as pltpu
from jax.experimental.pallas import tpu_sc as plsc

N = 10000
E = 160000
D = 128
HID = 64
NLAYER = 5
TASKS = 112
EPS = 1e-7

NC = 2
NS = 16
HD = D // NC
CH = 128
KSUB = CH // 128
NCHK = E // CH
HPAD = 16512
TB = 1000
EB = 4000


def _leaky(v):
    return jnp.where(v >= 0, v, 0.2 * v)


@functools.cache
def _build_edge_pass(base, rows):
    mesh = plsc.VectorSubcoreMesh(
        core_axis_name="c", subcore_axis_name="s",
        num_cores=NC, num_subcores=NS)
    return functools.partial(
        pl.kernel,
        out_type=jax.ShapeDtypeStruct((NC, rows, D), jnp.float32),
        mesh=mesh,
        scratch_types=[
            pltpu.VMEM((KSUB, 128), jnp.int32),
            pltpu.VMEM((KSUB, 128), jnp.int32),
            pltpu.VMEM((CH, D), jnp.float32),
            pltpu.VMEM((CH, HD), jnp.float32),
            pltpu.VMEM((16,), jnp.float32),
            pltpu.VMEM_SHARED((rows + NS, D), jnp.float32),
            pltpu.SemaphoreType.DMA,
        ],
    )(functools.partial(_edge_pass_body, base, rows))


def _edge_pass_body(base, rows, h, ee, src, dst, tvec, zeros, acc_out,
                    idxb, dstb, va, vb, tv, acc, sem):
    rps = (rows // NS) // 8 * 8
    tail = rows - NS * rps
    c = lax.axis_index("c")
    s = lax.axis_index("s")
    row0 = s * rps
    pltpu.sync_copy(zeros.at[pl.ds(row0, rps)], acc.at[pl.ds(row0, rps)])

    if tail:
        @pl.when(s == NS - 1)
        def _():
            tl0 = NS * rps
            pltpu.sync_copy(zeros.at[pl.ds(tl0, tail)],
                            acc.at[pl.ds(tl0, tail)])

    pltpu.sync_copy(tvec, tv)
    plsc.subcore_barrier()
    t = tv[:]

    nk = (NCHK - s + NS - 1) // NS

    def chunk(k, _):
        e0 = (s + k * NS) * CH
        for j in range(KSUB):
            pltpu.sync_copy(src.at[pl.ds(e0 + j * 128, 128)], idxb.at[j])
            pltpu.sync_copy(dst.at[pl.ds(e0 + j * 128, 128)], dstb.at[j])
        for j in range(KSUB):
            pltpu.async_copy(h.at[idxb.at[j]],
                             va.at[pl.ds(j * 128, 128)], sem).wait()
        pltpu.sync_copy(ee.at[c, pl.ds(e0, CH)], vb)

        if not (base == 0 and rows == N):
            for j in range(KSUB):
                for q in range(8):
                    sl = pl.ds(q * 16, 16)
                    dv = dstb[j, sl]
                    ok = (dv >= base) & (dv < base + rows)
                    dstb[j, sl] = jnp.where(ok, dv - base, rows + s)

        def make_edge(off):
            def edge(i, _):
                for q in range(HD // 16):
                    a = va[i, pl.ds(off + q * 16, 16)]
                    b = vb[i, pl.ds(q * 16, 16)]
                    m = jnp.maximum(a + b, 0.0) + EPS
                    ex = jnp.exp(m * t)
                    va[i, pl.ds(q * 16, 16)] = ex
                    va[i, pl.ds(HD + q * 16, 16)] = ex * m
                return 0
            return edge

        @pl.when(c == 0)
        def _():
            lax.fori_loop(0, CH, make_edge(0), 0)

        @pl.when(c == 1)
        def _():
            lax.fori_loop(0, CH, make_edge(HD), 0)

        for j in range(KSUB):
            pltpu.sync_copy(va.at[pl.ds(j * 128, 128)],
                            acc.at[dstb.at[j]], add=True)
        return 0

    lax.fori_loop(0, nk, chunk, 0)
    plsc.subcore_barrier()
    pltpu.sync_copy(acc.at[pl.ds(row0, rps)], acc_out.at[c, pl.ds(row0, rps)])

    if tail:
        @pl.when(s == NS - 1)
        def _():
            tl0 = NS * rps
            pltpu.sync_copy(acc.at[pl.ds(tl0, tail)],
                            acc_out.at[c, pl.ds(tl0, tail)])


def _embed_body(x_r, nf_r, woh_r, boh_r, wnf_r, bnf_r, w1_r, b1_r, w2_r, b2_r,
                h_r, cb_r):
    nf2 = jnp.dot(x_r[:], woh_r[:], preferred_element_type=jnp.float32) + boh_r[:]
    cat = jnp.concatenate([nf_r[:], nf2], axis=1)
    h0 = jnp.dot(cat, wnf_r[:], preferred_element_type=jnp.float32) + bnf_r[:]
    z = _leaky(jnp.dot(h0, w1_r[:], preferred_element_type=jnp.float32) + b1_r[:])
    nw = jax.nn.sigmoid(
        jnp.dot(z, w2_r[:], preferred_element_type=jnp.float32) + b2_r[:])
    h = h0 * nw
    h_r[:] = h
    cb_r[:] = h * nw


def _embed_call(x, nf, p):
    full = lambda shape: pl.BlockSpec(shape, lambda i: (0,) * len(shape))
    return pl.pallas_call(
        _embed_body,
        grid=(N // TB,),
        in_specs=[
            pl.BlockSpec((TB, 8), lambda i: (i, 0)),
            pl.BlockSpec((TB, 8), lambda i: (i, 0)),
            full((8, 8)), full((1, 8)),
            full((16, D)), full((1, D)),
            full((D, HID)), full((1, HID)),
            full((HID, 1)), full((1, 1)),
        ],
        out_specs=[pl.BlockSpec((TB, D), lambda i: (i, 0))] * 2,
        out_shape=[jax.ShapeDtypeStruct((HPAD, D), jnp.float32),
                   jax.ShapeDtypeStruct((N, D), jnp.float32)],
    )(x, nf, p["W_oh"], p["b_oh"].reshape(1, 8),
      p["W_nf"], p["b_nf"].reshape(1, D),
      p["lrn_W1"][0], p["lrn_b1"][0].reshape(1, HID),
      p["lrn_W2"][0], p["lrn_b2"][0].reshape(1, 1))


def _ee_body(ea_r, w_r, b_r, o_r):
    res = jnp.dot(ea_r[:], w_r[:], preferred_element_type=jnp.float32) + b_r[:]
    o_r[0] = res[:, :HD]
    o_r[1] = res[:, HD:]


def _ee_call(edge_attr, w, b):
    return pl.pallas_call(
        _ee_body,
        grid=(E // EB,),
        in_specs=[
            pl.BlockSpec((EB, 8), lambda i: (i, 0)),
            pl.BlockSpec((8, D), lambda i: (0, 0)),
            pl.BlockSpec((1, D), lambda i: (0, 0)),
        ],
        out_specs=pl.BlockSpec((NC, EB, HD), lambda i: (0, i, 0)),
        out_shape=jax.ShapeDtypeStruct((NC, E, HD), jnp.float32),
    )(edge_attr, w, b.reshape(1, D))


def _layer_body(h_r, cb_r, acc_r, w_r, b_r, g_r, bb_r,
                w1_r, b1_r, w2_r, b2_r, ho_r, co_r):
    agg0 = acc_r[0, :, HD:] / (acc_r[0, :, :HD] + 1e-16)
    agg1 = acc_r[1, :, HD:] / (acc_r[1, :, :HD] + 1e-16)
    u = h_r[:] + jnp.concatenate([agg0, agg1], axis=1)
    h1 = jnp.dot(u, w_r[:], preferred_element_type=jnp.float32) + b_r[:]
    mval = jnp.mean(h1, axis=-1, keepdims=True)
    dvl = h1 - mval
    var = jnp.mean(dvl * dvl, axis=-1, keepdims=True)
    hn = dvl / jnp.sqrt(var + 1e-5) * g_r[:] + bb_r[:]
    h2 = jnp.maximum(hn, 0.0)
    z = _leaky(jnp.dot(h2 + cb_r[:], w1_r[:],
                       preferred_element_type=jnp.float32) + b1_r[:])
    nw = jax.nn.sigmoid(
        jnp.dot(z, w2_r[:], preferred_element_type=jnp.float32) + b2_r[:])
    hf = h2 * nw
    ho_r[:] = hf
    co_r[:] = cb_r[:] + hf


def _layer_call(h, cb, accs, p, l):
    full = lambda shape: pl.BlockSpec(shape, lambda i: (0,) * len(shape))
    return pl.pallas_call(
        _layer_body,
        grid=(N // TB,),
        in_specs=[
            pl.BlockSpec((TB, D), lambda i: (i, 0)),
            pl.BlockSpec((TB, D), lambda i: (i, 0)),
            pl.BlockSpec((NC, TB, D), lambda i: (0, i, 0)),
            full((D, D)), full((1, D)), full((1, D)), full((1, D)),
            full((D, HID)), full((1, HID)),
            full((HID, 1)), full((1, 1)),
        ],
        out_specs=[pl.BlockSpec((TB, D), lambda i: (i, 0))] * 2,
        out_shape=[jax.ShapeDtypeStruct((HPAD, D), jnp.float32),
                   jax.ShapeDtypeStruct((N, D), jnp.float32)],
    )(h, cb, accs,
      p["gcn_W"][l], p["gcn_b"][l].reshape(1, D),
      p["ln_g"][l].reshape(1, D), p["ln_b"][l].reshape(1, D),
      p["lrn_W1"][l + 1], p["lrn_b1"][l + 1].reshape(1, HID),
      p["lrn_W2"][l + 1], p["lrn_b2"][l + 1].reshape(1, 1))


def _pred_body(cb_r, w_r, b_r, o_r):
    o_r[:] = jnp.dot(cb_r[:], w_r[:], preferred_element_type=jnp.float32) + b_r[:]


def _pred_call(cb, w, b):
    return pl.pallas_call(
        _pred_body,
        grid=(N // TB,),
        in_specs=[
            pl.BlockSpec((TB, D), lambda i: (i, 0)),
            pl.BlockSpec((D, TASKS), lambda i: (0, 0)),
            pl.BlockSpec((1, TASKS), lambda i: (0, 0)),
        ],
        out_specs=pl.BlockSpec((TB, TASKS), lambda i: (i, 0)),
        out_shape=jax.ShapeDtypeStruct((N, TASKS), jnp.float32),
    )(cb, w, b.reshape(1, TASKS))


def kernel(x, node_index, edge_index, edge_attr, params):
    p = params
    nf = jnp.take(p["node_features"], node_index, axis=0)
    h, cb = _embed_call(x, nf, p)
    ee = _ee_call(edge_attr, p["W_edge"], p["b_edge"])
    src = edge_index[0]
    dst = edge_index[1]
    zeros = jnp.zeros((HPAD, D), jnp.float32)
    for l in range(NLAYER):
        tvec = jnp.full((16,), p["gcn_t"][l], jnp.float32)
        accs = _build_edge_pass(0, N)(h, ee, src, dst, tvec, zeros)
        h, cb = _layer_call(h, cb, accs, p, l)
    return _pred_call(cb, p["W_pred"], p["b_pred"])
